# baseline (speedup 1.0000x reference)
import functools

import jax
import jax.numpy as jnp
from jax import lax
from jax.experimental import pallas as pl
from jax.experimental.pallas import tpu as pltpu
from jax.experimental.pallas import tpu_sc as plsc


def kernel(x, pos_table):
    B, T, D = x.shape
    Q, E = T // 8, D // 128

    info = plsc.get_sparse_core_info()
    NC, NS = info.num_cores, info.num_subcores
    NW = NC * NS

    t_per_w = T // NW
    CT = 8
    n_chunks = t_per_w // CT
    CHUNK = CT * D
    TD = T * D

    x_lin = x.reshape(B, Q, 8, E, 128).transpose(0, 1, 3, 2, 4).reshape(B * TD)
    pos_lin = (
        pos_table[:T].reshape(Q, 8, E, 128).transpose(0, 2, 1, 3).reshape(TD)
    )

    mesh = plsc.VectorSubcoreMesh(core_axis_name="c", subcore_axis_name="s")

    @functools.partial(
        pl.kernel,
        mesh=mesh,
        out_type=jax.ShapeDtypeStruct((B * TD,), jnp.float32),
        scratch_types=(
            [pltpu.VMEM((CHUNK,), jnp.float32) for _ in range(2 * B + 2)]
            + [pltpu.SemaphoreType.DMA for _ in range(4 * B + 2)]
        ),
    )
    def sc_add(x_hbm, pos_hbm, out_hbm, *bufs):
        xb = [bufs[:B], bufs[B:2 * B]]
        pb = bufs[2 * B:2 * B + 2]
        sems = bufs[2 * B + 2:]
        si = [sems[:B], sems[B:2 * B]]
        so = [sems[2 * B:3 * B], sems[3 * B:4 * B]]
        sp = sems[4 * B:4 * B + 2]

        wid = lax.axis_index("s") * NC + lax.axis_index("c")
        base = wid * (t_per_w * D)

        def x_off(c, b):
            return b * TD + base + c * CHUNK

        def start_in(c, b):
            pltpu.make_async_copy(
                x_hbm.at[pl.ds(x_off(c, b), CHUNK)], xb[c % 2][b], si[c % 2][b]
            ).start()

        def wait_in(c, b):
            pltpu.make_async_copy(
                x_hbm.at[pl.ds(x_off(c, b), CHUNK)], xb[c % 2][b], si[c % 2][b]
            ).wait()

        def start_out(c, b):
            pltpu.make_async_copy(
                xb[c % 2][b], out_hbm.at[pl.ds(x_off(c, b), CHUNK)], so[c % 2][b]
            ).start()

        def wait_out(c, b):
            pltpu.make_async_copy(
                xb[c % 2][b], out_hbm.at[pl.ds(x_off(c, b), CHUNK)], so[c % 2][b]
            ).wait()

        def start_pos(c):
            pltpu.make_async_copy(
                pos_hbm.at[pl.ds(base + c * CHUNK, CHUNK)], pb[c % 2], sp[c % 2]
            ).start()

        def wait_pos(c):
            pltpu.make_async_copy(
                pos_hbm.at[pl.ds(base + c * CHUNK, CHUNK)], pb[c % 2], sp[c % 2]
            ).wait()

        UJ = 2
        def compute(c):
            xbs, pos_v = xb[c % 2], pb[c % 2]
            def body(k, _):
                for jj in range(UJ):
                    s = pl.ds((k * UJ + jj) * 16, 16)
                    vp = pos_v[s]
                    for b in range(B):
                        xbs[b][s] = xbs[b][s] + vp
                return 0
            lax.fori_loop(0, CHUNK // (16 * UJ), body, 0)

        for b in range(B):
            start_in(0, b)
        start_pos(0)

        for c in range(n_chunks):
            if c + 1 < n_chunks:
                if c >= 1:
                    for b in range(B):
                        wait_out(c - 1, b)
                for b in range(B):
                    start_in(c + 1, b)
                start_pos(c + 1)
            wait_pos(c)
            for b in range(B):
                wait_in(c, b)
            compute(c)
            for b in range(B):
                start_out(c, b)

        for c in (n_chunks - 2, n_chunks - 1):
            for b in range(B):
                wait_out(c, b)

    out_lin = sc_add(x_lin, pos_lin)
    return (
        out_lin.reshape(B, Q, E, 8, 128)
        .transpose(0, 1, 3, 2, 4)
        .reshape(B, T, D)
    )


# speedup vs baseline: 1.0130x; 1.0130x over previous
import functools

import jax
import jax.numpy as jnp
from jax import lax
from jax.experimental import pallas as pl
from jax.experimental.pallas import tpu as pltpu
from jax.experimental.pallas import tpu_sc as plsc


def kernel(x, pos_table):
    B, T, D = x.shape
    Q, E = T // 8, D // 128

    info = plsc.get_sparse_core_info()
    NC, NS = info.num_cores, info.num_subcores
    NW = NC * NS

    t_per_w = T // NW
    CT = 16
    n_chunks = t_per_w // CT
    CHUNK = CT * D
    n_units = n_chunks * B
    NBUF = 4
    TD = T * D

    x_lin = x.reshape(B, Q, 8, E, 128).transpose(0, 1, 3, 2, 4).reshape(B * TD)
    pos_lin = (
        pos_table[:T].reshape(Q, 8, E, 128).transpose(0, 2, 1, 3).reshape(TD)
    )

    mesh = plsc.VectorSubcoreMesh(core_axis_name="c", subcore_axis_name="s")

    @functools.partial(
        pl.kernel,
        mesh=mesh,
        out_type=jax.ShapeDtypeStruct((B * TD,), jnp.float32),
        scratch_types=(
            [pltpu.VMEM((CHUNK,), jnp.float32) for _ in range(NBUF + 2)]
            + [pltpu.SemaphoreType.DMA for _ in range(2 * NBUF + 2)]
        ),
    )
    def sc_add(x_hbm, pos_hbm, out_hbm, *bufs):
        xb = bufs[:NBUF]
        pb = bufs[NBUF:NBUF + 2]
        si = bufs[NBUF + 2:2 * NBUF + 2]
        so = bufs[2 * NBUF + 2:3 * NBUF + 2]
        sp = bufs[3 * NBUF + 2:3 * NBUF + 4]

        wid = lax.axis_index("s") * NC + lax.axis_index("c")
        base = wid * (t_per_w * D)

        def x_off(u):
            c, b = divmod(u, B)
            return b * TD + base + c * CHUNK

        def start_in(u):
            pltpu.make_async_copy(
                x_hbm.at[pl.ds(x_off(u), CHUNK)], xb[u % NBUF], si[u % NBUF]
            ).start()

        def wait_in(u):
            pltpu.make_async_copy(
                x_hbm.at[pl.ds(x_off(u), CHUNK)], xb[u % NBUF], si[u % NBUF]
            ).wait()

        def start_out(u):
            pltpu.make_async_copy(
                xb[u % NBUF], out_hbm.at[pl.ds(x_off(u), CHUNK)], so[u % NBUF]
            ).start()

        def wait_out(u):
            pltpu.make_async_copy(
                xb[u % NBUF], out_hbm.at[pl.ds(x_off(u), CHUNK)], so[u % NBUF]
            ).wait()

        def start_pos(c):
            pltpu.make_async_copy(
                pos_hbm.at[pl.ds(base + c * CHUNK, CHUNK)], pb[c % 2], sp[c % 2]
            ).start()

        def wait_pos(c):
            pltpu.make_async_copy(
                pos_hbm.at[pl.ds(base + c * CHUNK, CHUNK)], pb[c % 2], sp[c % 2]
            ).wait()

        UNROLL = 8
        def accumulate(x_v, pos_v):
            def body(k, _):
                base_k = k * (16 * UNROLL)
                for j in range(UNROLL):
                    s = pl.ds(base_k + j * 16, 16)
                    plsc.addupdate(x_v.at[s], pos_v[s])
                return 0
            lax.fori_loop(0, CHUNK // (16 * UNROLL), body, 0)

        start_in(0)
        start_in(1)
        start_pos(0)

        for u in range(n_units):
            c, b = divmod(u, B)
            if b == 0:
                if c + 1 < n_chunks:
                    start_pos(c + 1)
                wait_pos(c)
            v = u + 2
            if v < n_units:
                if v >= NBUF:
                    wait_out(v - NBUF)
                start_in(v)
            wait_in(u)
            accumulate(xb[u % NBUF], pb[c % 2])
            start_out(u)

        for u in range(n_units - NBUF, n_units):
            wait_out(u)

    out_lin = sc_add(x_lin, pos_lin)
    return (
        out_lin.reshape(B, Q, E, 8, 128)
        .transpose(0, 1, 3, 2, 4)
        .reshape(B, T, D)
    )
